# baseline (speedup 1.0000x reference)
import functools

import jax
import jax.numpy as jnp
from jax import lax
from jax.experimental import pallas as pl
from jax.experimental.pallas import tpu as pltpu
from jax.experimental.pallas import tpu_sc as plsc

N = 10000
E = 320000
NW = 32
LIST_LEN = 11264
CHUNKS = LIST_LEN // 128
TABLE_ROWS = 314
CH = 8000
NCH = E // CH

_mesh = plsc.VectorSubcoreMesh(
    core_axis_name="c", subcore_axis_name="s", num_cores=2, num_subcores=16)


def _gathv(v, idx16):
    dn = lax.GatherDimensionNumbers(offset_dims=(), collapsed_slice_dims=(0,),
                                    start_index_map=(0,))
    return lax.gather(v, idx16.reshape(16, 1), dn, (1,),
                      mode=lax.GatherScatterMode.PROMISE_IN_BOUNDS)


def _bcast(v, lane):
    return _gathv(v, jnp.full((16,), lane, jnp.int32))


def _wid_lo_sz():
    wid = lax.axis_index("c") * 16 + lax.axis_index("s")
    lo = jnp.where(wid < 16, wid * 313, 5008 + (wid - 16) * 312)
    sz = jnp.where(wid < 16, 313, 312)
    return wid, lo, sz


def _bucket_body(src_hbm, dst_hbm, srcl_out, dll_out, srcb, dstb, pkl,
                 srcl, dll, offs):
    wid, lo, sz = _wid_lo_sz()
    hi = lo + sz
    lane = lax.iota(jnp.int32, 16)

    sentpk = jnp.full((16,), (TABLE_ROWS - 1) * 16384, jnp.int32)

    def fill(i, _):
        pkl[pl.ds(i * 16, 16)] = sentpk
        return 0
    lax.fori_loop(0, (LIST_LEN + 16) // 16, fill, 0)

    def zoffs(i, _):
        offs[pl.ds(i * 16, 16)] = jnp.zeros((16,), jnp.int32)
        return 0
    lax.fori_loop(0, 20, zoffs, 0)

    def chunk(c, cnt):
        pltpu.sync_copy(src_hbm.at[pl.ds(c * CH, CH)], srcb)
        pltpu.sync_copy(dst_hbm.at[pl.ds(c * CH, CH)], dstb)

        def inner(i, cnt):
            d16 = dstb[pl.ds(i * 16, 16)]
            s16 = srcb[pl.ds(i * 16, 16)]
            m = (d16 >= lo) & (d16 < hi)
            cs = plsc.cumsum(m.astype(jnp.int32))
            pos = cnt + cs - 1
            plsc.store_scatter(pkl, [pos], (d16 - lo) * 16384 + s16, mask=m)
            return cnt + _bcast(cs, 15)
        return lax.fori_loop(0, CH // 16, inner, cnt)

    lax.fori_loop(0, NCH, chunk, jnp.zeros((16,), jnp.int32))

    def hist(v, _):
        dl16 = pkl[pl.ds(v * 16, 16)] >> 14
        for i in range(16):
            iv = _bcast(dl16, i)
            c = plsc.load_gather(offs, [iv])
            plsc.store_scatter(offs, [iv], c + 1)
        return 0
    lax.fori_loop(0, LIST_LEN // 16, hist, 0)

    def pfx(v, carry):
        c16 = offs[pl.ds(v * 16, 16)]
        cs = plsc.cumsum(c16)
        offs[pl.ds(v * 16, 16)] = carry + cs - c16
        return carry + _bcast(cs, 15)
    lax.fori_loop(0, 20, pfx, jnp.zeros((16,), jnp.int32))

    l0 = lane == 0

    def place(v, _):
        pk16 = pkl[pl.ds(v * 16, 16)]
        dl16 = pk16 >> 14
        s16 = pk16 & 16383
        for i in range(16):
            iv = _bcast(dl16, i)
            p = plsc.load_gather(offs, [iv])
            plsc.store_scatter(offs, [iv], p + 1)
            plsc.store_scatter(srcl, [p], _bcast(s16, i), mask=l0)
            plsc.store_scatter(dll, [p], iv, mask=l0)
        return 0
    lax.fori_loop(0, LIST_LEN // 16, place, 0)

    pltpu.sync_copy(srcl.at[pl.ds(0, LIST_LEN)], srcl_out.at[wid])
    pltpu.sync_copy(dll.at[pl.ds(0, LIST_LEN)], dll_out.at[wid])


@functools.partial(
    pl.kernel,
    out_type=(jax.ShapeDtypeStruct((NW, LIST_LEN), jnp.int32),
              jax.ShapeDtypeStruct((NW, LIST_LEN), jnp.int32)),
    mesh=_mesh,
    scratch_types=[pltpu.VMEM((CH,), jnp.int32),
                   pltpu.VMEM((CH,), jnp.int32),
                   pltpu.VMEM((LIST_LEN + 16,), jnp.int32),
                   pltpu.VMEM((LIST_LEN + 16,), jnp.int32),
                   pltpu.VMEM((LIST_LEN + 16,), jnp.int32),
                   pltpu.VMEM((320,), jnp.int32)],
    compiler_params=pltpu.CompilerParams(needs_layout_passes=False,
                                         use_tc_tiling_on_sc=False),
)
def _bucketize(src_hbm, dst_hbm, srcl_out, dll_out, srcb, dstb, pkl,
               srcl, dll, offs):
    _bucket_body(src_hbm, dst_hbm, srcl_out, dll_out, srcb, dstb, pkl,
                 srcl, dll, offs)


def _segmax_body(b_hbm, srcl_hbm, dll_hbm, s_out, idx, dlb, rowsA, rowsB,
                 rowsC, rowsD, table, semA, semB, semC, semD):
    wid, lo, sz = _wid_lo_sz()
    pltpu.sync_copy(srcl_hbm.at[wid], idx)
    pltpu.sync_copy(dll_hbm.at[wid], dlb)

    neg16 = jnp.full((16,), -jnp.inf, jnp.float32)

    def initb(i, _):
        table[pl.ds(i * 16, 16)] = neg16
        return 0
    lax.fori_loop(0, TABLE_ROWS * 4, initb, 0)

    lane = lax.iota(jnp.int32, 16)
    shift_idx = jnp.maximum(lane - 1, 0)

    bufs = (rowsA, rowsB, rowsC, rowsD)
    sems = (semA, semB, semC, semD)
    for p in range(3):
        pltpu.async_copy(b_hbm.at[idx.at[p]], bufs[p], sems[p])

    def quad(j4, carry):
        for par in range(4):
            j = j4 * 4 + par
            rows = bufs[par]
            sem = sems[par]
            pltpu.make_async_copy(b_hbm.at[idx.at[j]], rows, sem).wait()

            @plsc.parallel_loop(0, 8, unroll=2, carry=carry)
            def grp(g, carry):
                prev, curb, a0, a1, a2, a3 = carry
                acc = [a0, a1, a2, a3]
                dl16 = dlb[pl.ds(j * 128 + g * 16, 16)]
                shifted = jnp.where(lane == 0, _bcast(prev, 15),
                                    _gathv(dl16, shift_idx))
                bst16 = (dl16 != shifted).astype(jnp.int32)
                base16 = dl16 * 64
                for i in range(16):
                    b = _bcast(bst16, i) != 0
                    nb = _bcast(base16, i) + lane
                    e = g * 16 + i
                    for f in range(4):
                        plsc.store_scatter(table, [curb + (f * 16)],
                                           acc[f], mask=b)
                        r = rows[e, pl.ds(f * 16, 16)]
                        acc[f] = jnp.where(b, r, jnp.maximum(acc[f], r))
                    curb = jnp.where(b, nb, curb)
                return (_bcast(dl16, 15), curb, acc[0], acc[1], acc[2], acc[3])

            carry = grp

            @pl.when(j < CHUNKS - 3)
            def _():
                pltpu.async_copy(b_hbm.at[idx.at[j + 3]],
                                 bufs[(par + 3) % 4], sems[(par + 3) % 4])
        return carry

    init = (jnp.full((16,), -1, jnp.int32),
            jnp.full((16,), (TABLE_ROWS - 1) * 64, jnp.int32) + lane,
            neg16, neg16, neg16, neg16)
    prev, curb, a0, a1, a2, a3 = lax.fori_loop(0, CHUNKS // 4, quad, init)
    for f, af in enumerate((a0, a1, a2, a3)):
        plsc.store_scatter(table, [curb + (f * 16)], af)

    @pl.when(wid < 16)
    def _():
        pltpu.sync_copy(table.at[pl.ds(0, 313 * 64)],
                        s_out.at[pl.ds(lo * 64, 313 * 64)])

    @pl.when(wid >= 16)
    def _():
        pltpu.sync_copy(table.at[pl.ds(0, 312 * 64)],
                        s_out.at[pl.ds(lo * 64, 312 * 64)])


@functools.partial(
    pl.kernel,
    out_type=jax.ShapeDtypeStruct((N * 64,), jnp.float32),
    mesh=_mesh,
    scratch_types=[pltpu.VMEM((CHUNKS, 128), jnp.int32),
                   pltpu.VMEM((LIST_LEN,), jnp.int32),
                   pltpu.VMEM((128, 64), jnp.float32),
                   pltpu.VMEM((128, 64), jnp.float32),
                   pltpu.VMEM((128, 64), jnp.float32),
                   pltpu.VMEM((128, 64), jnp.float32),
                   pltpu.VMEM((TABLE_ROWS * 64,), jnp.float32),
                   pltpu.SemaphoreType.DMA,
                   pltpu.SemaphoreType.DMA,
                   pltpu.SemaphoreType.DMA,
                   pltpu.SemaphoreType.DMA],
    compiler_params=pltpu.CompilerParams(needs_layout_passes=False,
                                         use_tc_tiling_on_sc=False),
)
def _segmax(b_hbm, srcl_hbm, dll_hbm, s_out, idx, dlb, rowsA, rowsB, rowsC,
            rowsD, table, semA, semB, semC, semD):
    _segmax_body(b_hbm, srcl_hbm, dll_hbm, s_out, idx, dlb, rowsA, rowsB,
                 rowsC, rowsD, table, semA, semB, semC, semD)


_R = 1000


def _lrelu(v):
    return jnp.where(v >= 0, v, 0.01 * v)


def _split_w(w, d):
    wa = w[:, :d]
    wb = w[:, d:]
    return jnp.concatenate([wa - wb, wb], axis=0)


def _tc1_body(x_ref, w_ref, b_ref, a_ref, bv_ref):
    wstk = _split_w(w_ref[...], 128)
    ab = lax.dot_general(x_ref[...], wstk, (((1,), (1,)), ((), ())),
                         preferred_element_type=jnp.float32)
    a_ref[...] = ab[:, :64] + b_ref[...]
    bv_ref[...] = ab[:, 64:]


def _tc1(x, w, b):
    return pl.pallas_call(
        _tc1_body,
        grid=(N // _R,),
        in_specs=[pl.BlockSpec((_R, 128), lambda i: (i, 0)),
                  pl.BlockSpec((64, 256), lambda i: (0, 0)),
                  pl.BlockSpec((1, 64), lambda i: (0, 0))],
        out_specs=[pl.BlockSpec((_R, 64), lambda i: (i, 0)),
                   pl.BlockSpec((_R, 64), lambda i: (i, 0))],
        out_shape=[jax.ShapeDtypeStruct((N, 64), jnp.float32),
                   jax.ShapeDtypeStruct((N, 64), jnp.float32)],
    )(x, w, b.reshape(1, 64))


def _make_layer_body(nx, d):
    def body(*refs):
        xrefs = refs[:nx]
        aprev, sprev, w_ref, b_ref = refs[nx:nx + 4]
        xout, aout, bvout = refs[nx + 4:]
        v = aprev[...] + sprev[...]
        xp = _lrelu(jnp.where(jnp.isfinite(v), v, 0.0))
        xout[...] = xp
        X = jnp.concatenate([r[...] for r in xrefs] + [xp], axis=1)
        wstk = _split_w(w_ref[...], d)
        ab = lax.dot_general(X, wstk, (((1,), (1,)), ((), ())),
                             preferred_element_type=jnp.float32)
        aout[...] = ab[:, :64] + b_ref[...]
        bvout[...] = ab[:, 64:]
    return body


def _tc_layer(xparts, aprev, sprev, w, b):
    widths = [int(p.shape[1]) for p in xparts]
    d = sum(widths) + 64
    in_specs = ([pl.BlockSpec((_R, wd), lambda i: (i, 0)) for wd in widths]
                + [pl.BlockSpec((_R, 64), lambda i: (i, 0)),
                   pl.BlockSpec((_R, 64), lambda i: (i, 0)),
                   pl.BlockSpec((64, 2 * d), lambda i: (0, 0)),
                   pl.BlockSpec((1, 64), lambda i: (0, 0))])
    return pl.pallas_call(
        _make_layer_body(len(xparts), d),
        grid=(N // _R,),
        in_specs=in_specs,
        out_specs=[pl.BlockSpec((_R, 64), lambda i: (i, 0))] * 3,
        out_shape=[jax.ShapeDtypeStruct((N, 64), jnp.float32)] * 3,
    )(*xparts, aprev, sprev, w, b.reshape(1, 64))


def _tcf_body(x0, x1, x2, x3, a4, s4, wm1, bm1, wm2, bm2, out):
    v = a4[...] + s4[...]
    x4 = _lrelu(jnp.where(jnp.isfinite(v), v, 0.0))
    F = jnp.concatenate([x0[...], x1[...], x2[...], x3[...], x4], axis=1)
    h = lax.dot_general(F, wm1[...], (((1,), (1,)), ((), ())),
                        preferred_element_type=jnp.float32) + bm1[...]
    out[...] = lax.dot_general(h, wm2[...], (((1,), (1,)), ((), ())),
                               preferred_element_type=jnp.float32) + bm2[...]


def _tc_final(x0, x1, x2, x3, a4, s4, wm1, bm1, wm2, bm2):
    return pl.pallas_call(
        _tcf_body,
        grid=(N // _R,),
        in_specs=[pl.BlockSpec((_R, 128), lambda i: (i, 0)),
                  pl.BlockSpec((_R, 64), lambda i: (i, 0)),
                  pl.BlockSpec((_R, 64), lambda i: (i, 0)),
                  pl.BlockSpec((_R, 64), lambda i: (i, 0)),
                  pl.BlockSpec((_R, 64), lambda i: (i, 0)),
                  pl.BlockSpec((_R, 64), lambda i: (i, 0)),
                  pl.BlockSpec((64, 384), lambda i: (0, 0)),
                  pl.BlockSpec((1, 64), lambda i: (0, 0)),
                  pl.BlockSpec((10, 64), lambda i: (0, 0)),
                  pl.BlockSpec((1, 10), lambda i: (0, 0))],
        out_specs=pl.BlockSpec((_R, 10), lambda i: (i, 0)),
        out_shape=jax.ShapeDtypeStruct((N, 10), jnp.float32),
    )(x0, x1, x2, x3, a4, s4, wm1, bm1.reshape(1, 64), wm2,
      bm2.reshape(1, 10))


def kernel(x, edge_index, W1, b1, W2, b2, W3, b3, W4, b4, Wm1, bm1, Wm2, bm2):
    src = edge_index[0]
    dst = edge_index[1]
    srcl, dll = _bucketize(src, dst)
    srcl3 = srcl.reshape(NW, CHUNKS, 128)

    a1, b1v = _tc1(x, W1, b1)
    s1 = _segmax(b1v, srcl3, dll).reshape(N, 64)
    x1, a2, b2v = _tc_layer([x], a1, s1, W2, b2)
    s2 = _segmax(b2v, srcl3, dll).reshape(N, 64)
    x2, a3, b3v = _tc_layer([x, x1], a2, s2, W3, b3)
    s3 = _segmax(b3v, srcl3, dll).reshape(N, 64)
    x3, a4, b4v = _tc_layer([x, x1, x2], a3, s3, W4, b4)
    s4 = _segmax(b4v, srcl3, dll).reshape(N, 64)
    return _tc_final(x, x1, x2, x3, a4, s4, Wm1, bm1, Wm2, bm2)


# speedup vs baseline: 1.7142x; 1.7142x over previous
import functools

import jax
import jax.numpy as jnp
from jax import lax
from jax.experimental import pallas as pl
from jax.experimental.pallas import tpu as pltpu
from jax.experimental.pallas import tpu_sc as plsc

N = 10000
E = 320000
NW = 32
LIST_LEN = 11264
CHUNKS = LIST_LEN // 128
TABLE_ROWS = 314
CH = 8000
NCH = E // CH

_mesh = plsc.VectorSubcoreMesh(
    core_axis_name="c", subcore_axis_name="s", num_cores=2, num_subcores=16)


def _gathv(v, idx16):
    dn = lax.GatherDimensionNumbers(offset_dims=(), collapsed_slice_dims=(0,),
                                    start_index_map=(0,))
    return lax.gather(v, idx16.reshape(16, 1), dn, (1,),
                      mode=lax.GatherScatterMode.PROMISE_IN_BOUNDS)


def _bcast(v, lane):
    return _gathv(v, jnp.full((16,), lane, jnp.int32))


def _wid_lo_sz():
    wid = lax.axis_index("c") * 16 + lax.axis_index("s")
    lo = jnp.where(wid < 16, wid * 313, 5008 + (wid - 16) * 312)
    sz = jnp.where(wid < 16, 313, 312)
    return wid, lo, sz


def _bucket_body(src_hbm, dst_hbm, srcl_out, dll_out, srcb, dstb, pkl,
                 srcl, dll, offs):
    wid, lo, sz = _wid_lo_sz()
    hi = lo + sz
    lane = lax.iota(jnp.int32, 16)

    sentpk = jnp.full((16,), (TABLE_ROWS - 1) * 16384, jnp.int32)

    def fill(i, _):
        pkl[pl.ds(i * 16, 16)] = sentpk
        return 0
    lax.fori_loop(0, (LIST_LEN + 16) // 16, fill, 0)

    def zoffs(i, _):
        offs[pl.ds(i * 16, 16)] = jnp.zeros((16,), jnp.int32)
        return 0
    lax.fori_loop(0, 20, zoffs, 0)

    def chunk(c, cnt):
        pltpu.sync_copy(src_hbm.at[pl.ds(c * CH, CH)], srcb)
        pltpu.sync_copy(dst_hbm.at[pl.ds(c * CH, CH)], dstb)

        def inner(i, cnt):
            d16 = dstb[pl.ds(i * 16, 16)]
            s16 = srcb[pl.ds(i * 16, 16)]
            m = (d16 >= lo) & (d16 < hi)
            cs = plsc.cumsum(m.astype(jnp.int32))
            pos = cnt + cs - 1
            plsc.store_scatter(pkl, [pos], (d16 - lo) * 16384 + s16, mask=m)
            return cnt + _bcast(cs, 15)
        return lax.fori_loop(0, CH // 16, inner, cnt)

    lax.fori_loop(0, NCH, chunk, jnp.zeros((16,), jnp.int32))

    def hist(v, _):
        dl16 = pkl[pl.ds(v * 16, 16)] >> 14
        for i in range(16):
            iv = _bcast(dl16, i)
            c = plsc.load_gather(offs, [iv])
            plsc.store_scatter(offs, [iv], c + 1)
        return 0
    lax.fori_loop(0, LIST_LEN // 16, hist, 0)

    def pfx(v, carry):
        c16 = offs[pl.ds(v * 16, 16)]
        cs = plsc.cumsum(c16)
        offs[pl.ds(v * 16, 16)] = carry + cs - c16
        return carry + _bcast(cs, 15)
    lax.fori_loop(0, 20, pfx, jnp.zeros((16,), jnp.int32))

    l0 = lane == 0

    def place(v, _):
        pk16 = pkl[pl.ds(v * 16, 16)]
        dl16 = pk16 >> 14
        s16 = pk16 & 16383
        for i in range(16):
            iv = _bcast(dl16, i)
            p = plsc.load_gather(offs, [iv])
            plsc.store_scatter(offs, [iv], p + 1)
            plsc.store_scatter(srcl, [p], _bcast(s16, i), mask=l0)
            plsc.store_scatter(dll, [p], iv, mask=l0)
        return 0
    lax.fori_loop(0, LIST_LEN // 16, place, 0)

    pltpu.sync_copy(srcl.at[pl.ds(0, LIST_LEN)], srcl_out.at[wid])
    pltpu.sync_copy(dll.at[pl.ds(0, LIST_LEN)], dll_out.at[wid])


@functools.partial(
    pl.kernel,
    out_type=(jax.ShapeDtypeStruct((NW, LIST_LEN), jnp.int32),
              jax.ShapeDtypeStruct((NW, LIST_LEN), jnp.int32)),
    mesh=_mesh,
    scratch_types=[pltpu.VMEM((CH,), jnp.int32),
                   pltpu.VMEM((CH,), jnp.int32),
                   pltpu.VMEM((LIST_LEN + 16,), jnp.int32),
                   pltpu.VMEM((LIST_LEN + 16,), jnp.int32),
                   pltpu.VMEM((LIST_LEN + 16,), jnp.int32),
                   pltpu.VMEM((320,), jnp.int32)],
    compiler_params=pltpu.CompilerParams(needs_layout_passes=False,
                                         use_tc_tiling_on_sc=False),
)
def _bucketize(src_hbm, dst_hbm, srcl_out, dll_out, srcb, dstb, pkl,
               srcl, dll, offs):
    _bucket_body(src_hbm, dst_hbm, srcl_out, dll_out, srcb, dstb, pkl,
                 srcl, dll, offs)


def _segmax_body(b_hbm, srcl_hbm, dll_hbm, s_out, idx, dlb, rowsA, rowsB,
                 rowsC, rowsD, table, semA, semB, semC, semD):
    wid, lo, sz = _wid_lo_sz()
    pltpu.sync_copy(srcl_hbm.at[wid], idx)
    pltpu.sync_copy(dll_hbm.at[wid], dlb)

    neg16 = jnp.full((16,), -jnp.inf, jnp.float32)

    def initb(i, _):
        table[pl.ds(i * 16, 16)] = neg16
        return 0
    lax.fori_loop(0, TABLE_ROWS * 4, initb, 0)

    lane = lax.iota(jnp.int32, 16)
    shift_idx = jnp.maximum(lane - 1, 0)

    bufs = (rowsA, rowsB, rowsC, rowsD)
    sems = (semA, semB, semC, semD)
    for p in range(3):
        pltpu.async_copy(b_hbm.at[idx.at[p]], bufs[p], sems[p])

    def quad(j4, carry):
        for par in range(4):
            j = j4 * 4 + par
            rows = bufs[par]
            sem = sems[par]
            pltpu.make_async_copy(b_hbm.at[idx.at[j]], rows, sem).wait()

            @plsc.parallel_loop(0, 8, unroll=2, carry=carry)
            def grp(g, carry):
                prev, curb, a0, a1, a2, a3 = carry
                acc = [a0, a1, a2, a3]
                dl16 = dlb[pl.ds(j * 128 + g * 16, 16)]
                shifted = jnp.where(lane == 0, _bcast(prev, 15),
                                    _gathv(dl16, shift_idx))
                bst16 = (dl16 != shifted).astype(jnp.int32)
                base16 = dl16 * 64
                for i in range(16):
                    b = _bcast(bst16, i) != 0
                    nb = _bcast(base16, i) + lane
                    e = g * 16 + i
                    rr = []
                    for f2 in range(2):
                        w32 = plsc.bitcast(rows[e, pl.ds(f2 * 32, 32)],
                                           jnp.int32)
                        rr.append(plsc.bitcast(w32 << 16, jnp.float32))
                        rr.append(plsc.bitcast(w32 & -65536, jnp.float32))
                    for f in range(4):
                        plsc.store_scatter(table, [curb + (f * 16)],
                                           acc[f], mask=b)
                        acc[f] = jnp.where(b, rr[f],
                                           jnp.maximum(acc[f], rr[f]))
                    curb = jnp.where(b, nb, curb)
                return (_bcast(dl16, 15), curb, acc[0], acc[1], acc[2], acc[3])

            carry = grp

            @pl.when(j < CHUNKS - 3)
            def _():
                pltpu.async_copy(b_hbm.at[idx.at[j + 3]],
                                 bufs[(par + 3) % 4], sems[(par + 3) % 4])
        return carry

    init = (jnp.full((16,), -1, jnp.int32),
            jnp.full((16,), (TABLE_ROWS - 1) * 64, jnp.int32) + lane,
            neg16, neg16, neg16, neg16)
    prev, curb, a0, a1, a2, a3 = lax.fori_loop(0, CHUNKS // 4, quad, init)
    for f, af in enumerate((a0, a1, a2, a3)):
        plsc.store_scatter(table, [curb + (f * 16)], af)

    @pl.when(wid < 16)
    def _():
        pltpu.sync_copy(table.at[pl.ds(0, 313 * 64)],
                        s_out.at[pl.ds(lo * 64, 313 * 64)])

    @pl.when(wid >= 16)
    def _():
        pltpu.sync_copy(table.at[pl.ds(0, 312 * 64)],
                        s_out.at[pl.ds(lo * 64, 312 * 64)])


@functools.partial(
    pl.kernel,
    out_type=jax.ShapeDtypeStruct((N * 64,), jnp.float32),
    mesh=_mesh,
    scratch_types=[pltpu.VMEM((CHUNKS, 128), jnp.int32),
                   pltpu.VMEM((LIST_LEN,), jnp.int32),
                   pltpu.VMEM((128, 64), jnp.bfloat16),
                   pltpu.VMEM((128, 64), jnp.bfloat16),
                   pltpu.VMEM((128, 64), jnp.bfloat16),
                   pltpu.VMEM((128, 64), jnp.bfloat16),
                   pltpu.VMEM((TABLE_ROWS * 64,), jnp.float32),
                   pltpu.SemaphoreType.DMA,
                   pltpu.SemaphoreType.DMA,
                   pltpu.SemaphoreType.DMA,
                   pltpu.SemaphoreType.DMA],
    compiler_params=pltpu.CompilerParams(needs_layout_passes=False,
                                         use_tc_tiling_on_sc=False),
)
def _segmax(b_hbm, srcl_hbm, dll_hbm, s_out, idx, dlb, rowsA, rowsB, rowsC,
            rowsD, table, semA, semB, semC, semD):
    _segmax_body(b_hbm, srcl_hbm, dll_hbm, s_out, idx, dlb, rowsA, rowsB,
                 rowsC, rowsD, table, semA, semB, semC, semD)


_R = 1000


def _lrelu(v):
    return jnp.where(v >= 0, v, 0.01 * v)


def _perm():
    import numpy as _np
    k = _np.arange(16)
    p01 = _np.ravel(_np.column_stack([k, k + 16]))
    return _np.concatenate([p01, p01 + 32])


def _prep_w(w, d):
    wa = w[:, :d]
    wb = w[:, d:]
    return jnp.concatenate([wa - wb, wb[_perm()]], axis=0)


def _tc1_body(x_ref, w_ref, b_ref, a_ref, bv_ref):
    ab = lax.dot_general(x_ref[...], w_ref[...], (((1,), (1,)), ((), ())),
                         preferred_element_type=jnp.float32)
    a_ref[...] = ab[:, :64] + b_ref[...]
    bv_ref[...] = ab[:, 64:].astype(jnp.bfloat16)


def _tc1(x, w, b):
    return pl.pallas_call(
        _tc1_body,
        grid=(N // _R,),
        in_specs=[pl.BlockSpec((_R, 128), lambda i: (i, 0)),
                  pl.BlockSpec((128, 128), lambda i: (0, 0)),
                  pl.BlockSpec((1, 64), lambda i: (0, 0))],
        out_specs=[pl.BlockSpec((_R, 64), lambda i: (i, 0)),
                   pl.BlockSpec((_R, 64), lambda i: (i, 0))],
        out_shape=[jax.ShapeDtypeStruct((N, 64), jnp.float32),
                   jax.ShapeDtypeStruct((N, 64), jnp.bfloat16)],
    )(x, _prep_w(w, 128), b.reshape(1, 64))


def _make_layer_body(nx, d):
    def body(*refs):
        xrefs = refs[:nx]
        aprev, sprev, w_ref, b_ref = refs[nx:nx + 4]
        xout, aout, bvout = refs[nx + 4:]
        v = aprev[...] + sprev[...]
        xp = _lrelu(jnp.where(jnp.isfinite(v), v, 0.0))
        xout[...] = xp
        X = jnp.concatenate([r[...] for r in xrefs] + [xp], axis=1)
        ab = lax.dot_general(X, w_ref[...], (((1,), (1,)), ((), ())),
                             preferred_element_type=jnp.float32)
        aout[...] = ab[:, :64] + b_ref[...]
        bvout[...] = ab[:, 64:].astype(jnp.bfloat16)
    return body


def _tc_layer(xparts, aprev, sprev, w, b):
    widths = [int(p.shape[1]) for p in xparts]
    d = sum(widths) + 64
    in_specs = ([pl.BlockSpec((_R, wd), lambda i: (i, 0)) for wd in widths]
                + [pl.BlockSpec((_R, 64), lambda i: (i, 0)),
                   pl.BlockSpec((_R, 64), lambda i: (i, 0)),
                   pl.BlockSpec((128, d), lambda i: (0, 0)),
                   pl.BlockSpec((1, 64), lambda i: (0, 0))])
    return pl.pallas_call(
        _make_layer_body(len(xparts), d),
        grid=(N // _R,),
        in_specs=in_specs,
        out_specs=[pl.BlockSpec((_R, 64), lambda i: (i, 0))] * 3,
        out_shape=[jax.ShapeDtypeStruct((N, 64), jnp.float32),
                   jax.ShapeDtypeStruct((N, 64), jnp.float32),
                   jax.ShapeDtypeStruct((N, 64), jnp.bfloat16)],
    )(*xparts, aprev, sprev, _prep_w(w, d), b.reshape(1, 64))


def _tcf_body(x0, x1, x2, x3, a4, s4, wm1, bm1, wm2, bm2, out):
    v = a4[...] + s4[...]
    x4 = _lrelu(jnp.where(jnp.isfinite(v), v, 0.0))
    F = jnp.concatenate([x0[...], x1[...], x2[...], x3[...], x4], axis=1)
    h = lax.dot_general(F, wm1[...], (((1,), (1,)), ((), ())),
                        preferred_element_type=jnp.float32) + bm1[...]
    out[...] = lax.dot_general(h, wm2[...], (((1,), (1,)), ((), ())),
                               preferred_element_type=jnp.float32) + bm2[...]


def _tc_final(x0, x1, x2, x3, a4, s4, wm1, bm1, wm2, bm2):
    return pl.pallas_call(
        _tcf_body,
        grid=(N // _R,),
        in_specs=[pl.BlockSpec((_R, 128), lambda i: (i, 0)),
                  pl.BlockSpec((_R, 64), lambda i: (i, 0)),
                  pl.BlockSpec((_R, 64), lambda i: (i, 0)),
                  pl.BlockSpec((_R, 64), lambda i: (i, 0)),
                  pl.BlockSpec((_R, 64), lambda i: (i, 0)),
                  pl.BlockSpec((_R, 64), lambda i: (i, 0)),
                  pl.BlockSpec((64, 384), lambda i: (0, 0)),
                  pl.BlockSpec((1, 64), lambda i: (0, 0)),
                  pl.BlockSpec((10, 64), lambda i: (0, 0)),
                  pl.BlockSpec((1, 10), lambda i: (0, 0))],
        out_specs=pl.BlockSpec((_R, 10), lambda i: (i, 0)),
        out_shape=jax.ShapeDtypeStruct((N, 10), jnp.float32),
    )(x0, x1, x2, x3, a4, s4, wm1, bm1.reshape(1, 64), wm2,
      bm2.reshape(1, 10))


def kernel(x, edge_index, W1, b1, W2, b2, W3, b3, W4, b4, Wm1, bm1, Wm2, bm2):
    src = edge_index[0]
    dst = edge_index[1]
    srcl, dll = _bucketize(src, dst)
    srcl3 = srcl.reshape(NW, CHUNKS, 128)

    a1, b1v = _tc1(x, W1, b1)
    s1 = _segmax(b1v, srcl3, dll).reshape(N, 64)
    x1, a2, b2v = _tc_layer([x], a1, s1, W2, b2)
    s2 = _segmax(b2v, srcl3, dll).reshape(N, 64)
    x2, a3, b3v = _tc_layer([x, x1], a2, s2, W3, b3)
    s3 = _segmax(b3v, srcl3, dll).reshape(N, 64)
    x3, a4, b4v = _tc_layer([x, x1, x2], a3, s3, W4, b4)
    s4 = _segmax(b4v, srcl3, dll).reshape(N, 64)
    return _tc_final(x, x1, x2, x3, a4, s4, Wm1, bm1, Wm2, bm2)


# speedup vs baseline: 2.0182x; 1.1773x over previous
import functools

import jax
import jax.numpy as jnp
from jax import lax
from jax.experimental import pallas as pl
from jax.experimental.pallas import tpu as pltpu
from jax.experimental.pallas import tpu_sc as plsc

N = 10000
E = 320000
NW = 32
LIST_LEN = 11264
CHUNKS = LIST_LEN // 128
TABLE_ROWS = 314
CH = 8000
NCH = E // CH

_mesh = plsc.VectorSubcoreMesh(
    core_axis_name="c", subcore_axis_name="s", num_cores=2, num_subcores=16)


def _gathv(v, idx16):
    dn = lax.GatherDimensionNumbers(offset_dims=(), collapsed_slice_dims=(0,),
                                    start_index_map=(0,))
    return lax.gather(v, idx16.reshape(16, 1), dn, (1,),
                      mode=lax.GatherScatterMode.PROMISE_IN_BOUNDS)


def _bcast(v, lane):
    return _gathv(v, jnp.full((16,), lane, jnp.int32))


def _wid_lo_sz():
    wid = lax.axis_index("c") * 16 + lax.axis_index("s")
    lo = wid * 313
    sz = jnp.where(wid < 31, 313, 297)
    return wid, lo, sz


EPW = E // NW
BSLOT = 1024
SENT_PK = (TABLE_ROWS - 1) * 16384


def _phase_a_body(src_hbm, dst_hbm, inter_out, srcb, dstb, stag, cnt2, sem):
    wid, _, _ = _wid_lo_sz()
    lane = lax.iota(jnp.int32, 16)
    sent16 = jnp.full((16,), SENT_PK, jnp.int32)

    def fill(i, _):
        stag[pl.ds(i * 16, 16)] = sent16
        return 0
    lax.fori_loop(0, NW * BSLOT // 16, fill, 0)

    def zc(i, _):
        cnt2[pl.ds(i * 16, 16)] = jnp.zeros((16,), jnp.int32)
        return 0
    lax.fori_loop(0, NW, zc, 0)

    pltpu.sync_copy(src_hbm.at[pl.ds(wid * EPW, EPW)], srcb)
    pltpu.sync_copy(dst_hbm.at[pl.ds(wid * EPW, EPW)], dstb)

    def scan(i, _):
        d16 = dstb[pl.ds(i * 16, 16)]
        s16 = srcb[pl.ds(i * 16, 16)]
        bkt = d16 // 313
        pk = (d16 - bkt * 313) * 16384 + s16
        idxv = bkt * 16 + lane
        p = plsc.load_gather(cnt2, [idxv])
        plsc.store_scatter(cnt2, [idxv], p + 1)
        plsc.store_scatter(stag, [bkt * BSLOT + lane * 64 + p], pk)
        return 0
    lax.fori_loop(0, EPW // 16, scan, 0)

    for b in range(NW):
        pltpu.async_copy(stag.at[pl.ds(b * BSLOT, BSLOT)],
                         inter_out.at[pl.ds((wid * NW + b) * BSLOT, BSLOT)],
                         sem)
    for b in range(NW):
        pltpu.make_async_copy(stag.at[pl.ds(b * BSLOT, BSLOT)],
                              inter_out.at[pl.ds((wid * NW + b) * BSLOT,
                                                 BSLOT)], sem).wait()


@functools.partial(
    pl.kernel,
    out_type=jax.ShapeDtypeStruct((NW * NW * BSLOT,), jnp.int32),
    mesh=_mesh,
    scratch_types=[pltpu.VMEM((EPW,), jnp.int32),
                   pltpu.VMEM((EPW,), jnp.int32),
                   pltpu.VMEM((NW * BSLOT,), jnp.int32),
                   pltpu.VMEM((NW * 16,), jnp.int32),
                   pltpu.SemaphoreType.DMA],
    compiler_params=pltpu.CompilerParams(needs_layout_passes=False,
                                         use_tc_tiling_on_sc=False),
)
def _phase_a(src_hbm, dst_hbm, inter_out, srcb, dstb, stag, cnt2, sem):
    _phase_a_body(src_hbm, dst_hbm, inter_out, srcb, dstb, stag, cnt2, sem)


def _bucket_body(inter_hbm, srcl_out, dll_out, gath, srcl, dll, cnth, sem):
    wid, lo, sz = _wid_lo_sz()
    lane = lax.iota(jnp.int32, 16)
    GV = NW * BSLOT // 16

    for t in range(NW):
        pltpu.async_copy(inter_hbm.at[pl.ds((t * NW + wid) * BSLOT, BSLOT)],
                         gath.at[pl.ds(t * BSLOT, BSLOT)], sem)

    zero16 = jnp.zeros((16,), jnp.int32)
    sent16 = jnp.full((16,), TABLE_ROWS - 1, jnp.int32)

    def fill(i, _):
        srcl[pl.ds(i * 16, 16)] = zero16
        dll[pl.ds(i * 16, 16)] = sent16
        return 0
    lax.fori_loop(0, (LIST_LEN + 16) // 16, fill, 0)

    def zc(i, _):
        cnth[pl.ds(i * 16, 16)] = zero16
        return 0
    lax.fori_loop(0, (TABLE_ROWS + 1) * 16 // 16, zc, 0)

    for t in range(NW):
        pltpu.make_async_copy(inter_hbm.at[pl.ds((t * NW + wid) * BSLOT,
                                                 BSLOT)],
                              gath.at[pl.ds(t * BSLOT, BSLOT)], sem).wait()

    def hist(v, _):
        dl16 = gath[pl.ds(v * 16, 16)] >> 14
        idxv = dl16 * 16 + lane
        c = plsc.load_gather(cnth, [idxv])
        plsc.store_scatter(cnth, [idxv], c + 1)
        return 0
    lax.fori_loop(0, GV, hist, 0)

    def pfx(v, carry):
        c16 = cnth[pl.ds(v * 16, 16)]
        cs = plsc.cumsum(c16)
        cnth[pl.ds(v * 16, 16)] = carry + cs - c16
        return carry + _bcast(cs, 15)
    lax.fori_loop(0, TABLE_ROWS * 16 // 16, pfx, jnp.zeros((16,), jnp.int32))

    def place(v, _):
        pk16 = gath[pl.ds(v * 16, 16)]
        dl16 = pk16 >> 14
        idxv = dl16 * 16 + lane
        p = plsc.load_gather(cnth, [idxv])
        plsc.store_scatter(cnth, [idxv], p + 1)
        m = dl16 < (TABLE_ROWS - 1)
        plsc.store_scatter(srcl, [p], pk16 & 16383, mask=m)
        plsc.store_scatter(dll, [p], dl16, mask=m)
        return 0
    lax.fori_loop(0, GV, place, 0)

    pltpu.sync_copy(srcl.at[pl.ds(0, LIST_LEN)], srcl_out.at[wid])
    pltpu.sync_copy(dll.at[pl.ds(0, LIST_LEN)], dll_out.at[wid])


@functools.partial(
    pl.kernel,
    out_type=(jax.ShapeDtypeStruct((NW, LIST_LEN), jnp.int32),
              jax.ShapeDtypeStruct((NW, LIST_LEN), jnp.int32)),
    mesh=_mesh,
    scratch_types=[pltpu.VMEM((NW * BSLOT,), jnp.int32),
                   pltpu.VMEM((LIST_LEN + 16,), jnp.int32),
                   pltpu.VMEM((LIST_LEN + 16,), jnp.int32),
                   pltpu.VMEM(((TABLE_ROWS + 1) * 16,), jnp.int32),
                   pltpu.SemaphoreType.DMA],
    compiler_params=pltpu.CompilerParams(needs_layout_passes=False,
                                         use_tc_tiling_on_sc=False),
)
def _bucketize(inter_hbm, srcl_out, dll_out, gath, srcl, dll, cnth, sem):
    _bucket_body(inter_hbm, srcl_out, dll_out, gath, srcl, dll, cnth, sem)


def _segmax_body(b_hbm, srcl_hbm, dll_hbm, s_out, idx, dlb, rowsA, rowsB,
                 rowsC, rowsD, table, semA, semB, semC, semD):
    wid, lo, sz = _wid_lo_sz()
    pltpu.sync_copy(srcl_hbm.at[wid], idx)
    pltpu.sync_copy(dll_hbm.at[wid], dlb)

    neg16 = jnp.full((16,), -jnp.inf, jnp.float32)

    def initb(i, _):
        table[pl.ds(i * 16, 16)] = neg16
        return 0
    lax.fori_loop(0, TABLE_ROWS * 4, initb, 0)

    lane = lax.iota(jnp.int32, 16)
    shift_idx = jnp.maximum(lane - 1, 0)

    bufs = (rowsA, rowsB, rowsC, rowsD)
    sems = (semA, semB, semC, semD)
    for p in range(3):
        pltpu.async_copy(b_hbm.at[idx.at[p]], bufs[p], sems[p])

    def quad(j4, carry):
        for par in range(4):
            j = j4 * 4 + par
            rows = bufs[par]
            sem = sems[par]
            pltpu.make_async_copy(b_hbm.at[idx.at[j]], rows, sem).wait()

            @plsc.parallel_loop(0, 8, unroll=2, carry=carry)
            def grp(g, carry):
                prev, curb, a0, a1, a2, a3 = carry
                acc = [a0, a1, a2, a3]
                dl16 = dlb[pl.ds(j * 128 + g * 16, 16)]
                shifted = jnp.where(lane == 0, _bcast(prev, 15),
                                    _gathv(dl16, shift_idx))
                bst16 = (dl16 != shifted).astype(jnp.int32)
                base16 = dl16 * 64
                for i in range(16):
                    b = _bcast(bst16, i) != 0
                    nb = _bcast(base16, i) + lane
                    e = g * 16 + i
                    rr = []
                    for f2 in range(2):
                        w32 = plsc.bitcast(rows[e, pl.ds(f2 * 32, 32)],
                                           jnp.int32)
                        rr.append(plsc.bitcast(w32 << 16, jnp.float32))
                        rr.append(plsc.bitcast(w32 & -65536, jnp.float32))
                    for f in range(4):
                        plsc.store_scatter(table, [curb + (f * 16)],
                                           acc[f], mask=b)
                        acc[f] = jnp.where(b, rr[f],
                                           jnp.maximum(acc[f], rr[f]))
                    curb = jnp.where(b, nb, curb)
                return (_bcast(dl16, 15), curb, acc[0], acc[1], acc[2], acc[3])

            carry = grp

            @pl.when(j < CHUNKS - 3)
            def _():
                pltpu.async_copy(b_hbm.at[idx.at[j + 3]],
                                 bufs[(par + 3) % 4], sems[(par + 3) % 4])
        return carry

    init = (jnp.full((16,), -1, jnp.int32),
            jnp.full((16,), (TABLE_ROWS - 1) * 64, jnp.int32) + lane,
            neg16, neg16, neg16, neg16)
    prev, curb, a0, a1, a2, a3 = lax.fori_loop(0, CHUNKS // 4, quad, init)
    for f, af in enumerate((a0, a1, a2, a3)):
        plsc.store_scatter(table, [curb + (f * 16)], af)

    @pl.when(wid < 31)
    def _():
        pltpu.sync_copy(table.at[pl.ds(0, 313 * 64)],
                        s_out.at[pl.ds(lo * 64, 313 * 64)])

    @pl.when(wid == 31)
    def _():
        pltpu.sync_copy(table.at[pl.ds(0, 297 * 64)],
                        s_out.at[pl.ds(lo * 64, 297 * 64)])


@functools.partial(
    pl.kernel,
    out_type=jax.ShapeDtypeStruct((N * 64,), jnp.float32),
    mesh=_mesh,
    scratch_types=[pltpu.VMEM((CHUNKS, 128), jnp.int32),
                   pltpu.VMEM((LIST_LEN,), jnp.int32),
                   pltpu.VMEM((128, 64), jnp.bfloat16),
                   pltpu.VMEM((128, 64), jnp.bfloat16),
                   pltpu.VMEM((128, 64), jnp.bfloat16),
                   pltpu.VMEM((128, 64), jnp.bfloat16),
                   pltpu.VMEM((TABLE_ROWS * 64,), jnp.float32),
                   pltpu.SemaphoreType.DMA,
                   pltpu.SemaphoreType.DMA,
                   pltpu.SemaphoreType.DMA,
                   pltpu.SemaphoreType.DMA],
    compiler_params=pltpu.CompilerParams(needs_layout_passes=False,
                                         use_tc_tiling_on_sc=False),
)
def _segmax(b_hbm, srcl_hbm, dll_hbm, s_out, idx, dlb, rowsA, rowsB, rowsC,
            rowsD, table, semA, semB, semC, semD):
    _segmax_body(b_hbm, srcl_hbm, dll_hbm, s_out, idx, dlb, rowsA, rowsB,
                 rowsC, rowsD, table, semA, semB, semC, semD)


_R = 1000


def _lrelu(v):
    return jnp.where(v >= 0, v, 0.01 * v)


def _perm():
    import numpy as _np
    k = _np.arange(16)
    p01 = _np.ravel(_np.column_stack([k, k + 16]))
    return _np.concatenate([p01, p01 + 32])


def _prep_w(w, d):
    wa = w[:, :d]
    wb = w[:, d:]
    return jnp.concatenate([wa - wb, wb[_perm()]], axis=0)


def _tc1_body(x_ref, w_ref, b_ref, a_ref, bv_ref):
    ab = lax.dot_general(x_ref[...], w_ref[...], (((1,), (1,)), ((), ())),
                         preferred_element_type=jnp.float32)
    a_ref[...] = ab[:, :64] + b_ref[...]
    bv_ref[...] = ab[:, 64:].astype(jnp.bfloat16)


def _tc1(x, w, b):
    return pl.pallas_call(
        _tc1_body,
        grid=(N // _R,),
        in_specs=[pl.BlockSpec((_R, 128), lambda i: (i, 0)),
                  pl.BlockSpec((128, 128), lambda i: (0, 0)),
                  pl.BlockSpec((1, 64), lambda i: (0, 0))],
        out_specs=[pl.BlockSpec((_R, 64), lambda i: (i, 0)),
                   pl.BlockSpec((_R, 64), lambda i: (i, 0))],
        out_shape=[jax.ShapeDtypeStruct((N, 64), jnp.float32),
                   jax.ShapeDtypeStruct((N, 64), jnp.bfloat16)],
    )(x, _prep_w(w, 128), b.reshape(1, 64))


def _make_layer_body(nx, d):
    def body(*refs):
        xrefs = refs[:nx]
        aprev, sprev, w_ref, b_ref = refs[nx:nx + 4]
        xout, aout, bvout = refs[nx + 4:]
        v = aprev[...] + sprev[...]
        xp = _lrelu(jnp.where(jnp.isfinite(v), v, 0.0))
        xout[...] = xp
        X = jnp.concatenate([r[...] for r in xrefs] + [xp], axis=1)
        ab = lax.dot_general(X, w_ref[...], (((1,), (1,)), ((), ())),
                             preferred_element_type=jnp.float32)
        aout[...] = ab[:, :64] + b_ref[...]
        bvout[...] = ab[:, 64:].astype(jnp.bfloat16)
    return body


def _tc_layer(xparts, aprev, sprev, w, b):
    widths = [int(p.shape[1]) for p in xparts]
    d = sum(widths) + 64
    in_specs = ([pl.BlockSpec((_R, wd), lambda i: (i, 0)) for wd in widths]
                + [pl.BlockSpec((_R, 64), lambda i: (i, 0)),
                   pl.BlockSpec((_R, 64), lambda i: (i, 0)),
                   pl.BlockSpec((128, d), lambda i: (0, 0)),
                   pl.BlockSpec((1, 64), lambda i: (0, 0))])
    return pl.pallas_call(
        _make_layer_body(len(xparts), d),
        grid=(N // _R,),
        in_specs=in_specs,
        out_specs=[pl.BlockSpec((_R, 64), lambda i: (i, 0))] * 3,
        out_shape=[jax.ShapeDtypeStruct((N, 64), jnp.float32),
                   jax.ShapeDtypeStruct((N, 64), jnp.float32),
                   jax.ShapeDtypeStruct((N, 64), jnp.bfloat16)],
    )(*xparts, aprev, sprev, _prep_w(w, d), b.reshape(1, 64))


def _tcf_body(x0, x1, x2, x3, a4, s4, wm1, bm1, wm2, bm2, out):
    v = a4[...] + s4[...]
    x4 = _lrelu(jnp.where(jnp.isfinite(v), v, 0.0))
    F = jnp.concatenate([x0[...], x1[...], x2[...], x3[...], x4], axis=1)
    h = lax.dot_general(F, wm1[...], (((1,), (1,)), ((), ())),
                        preferred_element_type=jnp.float32) + bm1[...]
    out[...] = lax.dot_general(h, wm2[...], (((1,), (1,)), ((), ())),
                               preferred_element_type=jnp.float32) + bm2[...]


def _tc_final(x0, x1, x2, x3, a4, s4, wm1, bm1, wm2, bm2):
    return pl.pallas_call(
        _tcf_body,
        grid=(N // _R,),
        in_specs=[pl.BlockSpec((_R, 128), lambda i: (i, 0)),
                  pl.BlockSpec((_R, 64), lambda i: (i, 0)),
                  pl.BlockSpec((_R, 64), lambda i: (i, 0)),
                  pl.BlockSpec((_R, 64), lambda i: (i, 0)),
                  pl.BlockSpec((_R, 64), lambda i: (i, 0)),
                  pl.BlockSpec((_R, 64), lambda i: (i, 0)),
                  pl.BlockSpec((64, 384), lambda i: (0, 0)),
                  pl.BlockSpec((1, 64), lambda i: (0, 0)),
                  pl.BlockSpec((10, 64), lambda i: (0, 0)),
                  pl.BlockSpec((1, 10), lambda i: (0, 0))],
        out_specs=pl.BlockSpec((_R, 10), lambda i: (i, 0)),
        out_shape=jax.ShapeDtypeStruct((N, 10), jnp.float32),
    )(x0, x1, x2, x3, a4, s4, wm1, bm1.reshape(1, 64), wm2,
      bm2.reshape(1, 10))


def kernel(x, edge_index, W1, b1, W2, b2, W3, b3, W4, b4, Wm1, bm1, Wm2, bm2):
    src = edge_index[0]
    dst = edge_index[1]
    inter = _phase_a(src, dst)
    srcl, dll = _bucketize(inter)
    srcl3 = srcl.reshape(NW, CHUNKS, 128)

    a1, b1v = _tc1(x, W1, b1)
    s1 = _segmax(b1v, srcl3, dll).reshape(N, 64)
    x1, a2, b2v = _tc_layer([x], a1, s1, W2, b2)
    s2 = _segmax(b2v, srcl3, dll).reshape(N, 64)
    x2, a3, b3v = _tc_layer([x, x1], a2, s2, W3, b3)
    s3 = _segmax(b3v, srcl3, dll).reshape(N, 64)
    x3, a4, b4v = _tc_layer([x, x1, x2], a3, s3, W4, b4)
    s4 = _segmax(b4v, srcl3, dll).reshape(N, 64)
    return _tc_final(x, x1, x2, x3, a4, s4, Wm1, bm1, Wm2, bm2)


# speedup vs baseline: 2.9268x; 1.4502x over previous
import functools

import jax
import jax.numpy as jnp
from jax import lax
from jax.experimental import pallas as pl
from jax.experimental.pallas import tpu as pltpu
from jax.experimental.pallas import tpu_sc as plsc

N = 10000
E = 320000
NW = 32
LIST_LEN = 10752
CHUNKS = LIST_LEN // 128
TABLE_ROWS = 314
CH = 8000
NCH = E // CH

_mesh = plsc.VectorSubcoreMesh(
    core_axis_name="c", subcore_axis_name="s", num_cores=2, num_subcores=16)


def _gathv(v, idx16):
    dn = lax.GatherDimensionNumbers(offset_dims=(), collapsed_slice_dims=(0,),
                                    start_index_map=(0,))
    return lax.gather(v, idx16.reshape(16, 1), dn, (1,),
                      mode=lax.GatherScatterMode.PROMISE_IN_BOUNDS)


def _bcast(v, lane):
    return _gathv(v, jnp.full((16,), lane, jnp.int32))


def _wid_lo_sz():
    wid = lax.axis_index("c") * 16 + lax.axis_index("s")
    lo = wid * 313
    sz = jnp.where(wid < 31, 313, 297)
    return wid, lo, sz


EPW = E // NW
BSLOT = 1024
SENT_PK = (TABLE_ROWS - 1) * 16384


def _phase_a_body(src_hbm, dst_hbm, inter_out, srcb, dstb, stag, cnt2, sem):
    wid, _, _ = _wid_lo_sz()
    lane = lax.iota(jnp.int32, 16)
    sent16 = jnp.full((16,), SENT_PK, jnp.int32)

    def fill(i, _):
        stag[pl.ds(i * 16, 16)] = sent16
        return 0
    lax.fori_loop(0, NW * BSLOT // 16, fill, 0)

    def zc(i, _):
        cnt2[pl.ds(i * 16, 16)] = jnp.zeros((16,), jnp.int32)
        return 0
    lax.fori_loop(0, NW, zc, 0)

    pltpu.sync_copy(src_hbm.at[pl.ds(wid * EPW, EPW)], srcb)
    pltpu.sync_copy(dst_hbm.at[pl.ds(wid * EPW, EPW)], dstb)

    def scan(i, _):
        d16 = dstb[pl.ds(i * 16, 16)]
        s16 = srcb[pl.ds(i * 16, 16)]
        bkt = d16 // 313
        pk = (d16 - bkt * 313) * 16384 + s16
        idxv = bkt * 16 + lane
        p = plsc.load_gather(cnt2, [idxv])
        plsc.store_scatter(cnt2, [idxv], p + 1)
        plsc.store_scatter(stag, [bkt * BSLOT + lane * 64 + p], pk)
        return 0
    lax.fori_loop(0, EPW // 16, scan, 0)

    for b in range(NW):
        pltpu.async_copy(stag.at[pl.ds(b * BSLOT, BSLOT)],
                         inter_out.at[pl.ds((wid * NW + b) * BSLOT, BSLOT)],
                         sem)
    for b in range(NW):
        pltpu.make_async_copy(stag.at[pl.ds(b * BSLOT, BSLOT)],
                              inter_out.at[pl.ds((wid * NW + b) * BSLOT,
                                                 BSLOT)], sem).wait()


@functools.partial(
    pl.kernel,
    out_type=jax.ShapeDtypeStruct((NW * NW * BSLOT,), jnp.int32),
    mesh=_mesh,
    scratch_types=[pltpu.VMEM((EPW,), jnp.int32),
                   pltpu.VMEM((EPW,), jnp.int32),
                   pltpu.VMEM((NW * BSLOT,), jnp.int32),
                   pltpu.VMEM((NW * 16,), jnp.int32),
                   pltpu.SemaphoreType.DMA],
    compiler_params=pltpu.CompilerParams(needs_layout_passes=False,
                                         use_tc_tiling_on_sc=False),
)
def _phase_a(src_hbm, dst_hbm, inter_out, srcb, dstb, stag, cnt2, sem):
    _phase_a_body(src_hbm, dst_hbm, inter_out, srcb, dstb, stag, cnt2, sem)


def _bucket_body(inter_hbm, srcl_out, dll_out, gath, srcl, dll, cnth, sem):
    wid, lo, sz = _wid_lo_sz()
    lane = lax.iota(jnp.int32, 16)
    GV = NW * BSLOT // 16

    for t in range(NW):
        pltpu.async_copy(inter_hbm.at[pl.ds((t * NW + wid) * BSLOT, BSLOT)],
                         gath.at[pl.ds(t * BSLOT, BSLOT)], sem)

    zero16 = jnp.zeros((16,), jnp.int32)
    sent16 = jnp.full((16,), TABLE_ROWS - 1, jnp.int32)

    def fill(i, _):
        srcl[pl.ds(i * 16, 16)] = zero16
        dll[pl.ds(i * 16, 16)] = sent16
        return 0
    lax.fori_loop(0, (LIST_LEN + 16) // 16, fill, 0)

    def zc(i, _):
        cnth[pl.ds(i * 16, 16)] = zero16
        return 0
    lax.fori_loop(0, (TABLE_ROWS + 1) * 16 // 16, zc, 0)

    for t in range(NW):
        pltpu.make_async_copy(inter_hbm.at[pl.ds((t * NW + wid) * BSLOT,
                                                 BSLOT)],
                              gath.at[pl.ds(t * BSLOT, BSLOT)], sem).wait()

    def hist(v, _):
        dl16 = gath[pl.ds(v * 16, 16)] >> 14
        idxv = dl16 * 16 + lane
        c = plsc.load_gather(cnth, [idxv])
        plsc.store_scatter(cnth, [idxv], c + 1)
        return 0
    lax.fori_loop(0, GV, hist, 0)

    def pfx(v, carry):
        c16 = cnth[pl.ds(v * 16, 16)]
        cs = plsc.cumsum(c16)
        cnth[pl.ds(v * 16, 16)] = carry + cs - c16
        return carry + _bcast(cs, 15)
    lax.fori_loop(0, TABLE_ROWS * 16 // 16, pfx, jnp.zeros((16,), jnp.int32))

    def place(v, _):
        pk16 = gath[pl.ds(v * 16, 16)]
        dl16 = pk16 >> 14
        idxv = dl16 * 16 + lane
        p = plsc.load_gather(cnth, [idxv])
        plsc.store_scatter(cnth, [idxv], p + 1)
        m = dl16 < (TABLE_ROWS - 1)
        plsc.store_scatter(srcl, [p], pk16 & 16383, mask=m)
        plsc.store_scatter(dll, [p], dl16, mask=m)
        return 0
    lax.fori_loop(0, GV, place, 0)

    pltpu.sync_copy(srcl.at[pl.ds(0, LIST_LEN)], srcl_out.at[wid])
    pltpu.sync_copy(dll.at[pl.ds(0, LIST_LEN)], dll_out.at[wid])


@functools.partial(
    pl.kernel,
    out_type=(jax.ShapeDtypeStruct((NW, LIST_LEN), jnp.int32),
              jax.ShapeDtypeStruct((NW, LIST_LEN), jnp.int32)),
    mesh=_mesh,
    scratch_types=[pltpu.VMEM((NW * BSLOT,), jnp.int32),
                   pltpu.VMEM((LIST_LEN + 16,), jnp.int32),
                   pltpu.VMEM((LIST_LEN + 16,), jnp.int32),
                   pltpu.VMEM(((TABLE_ROWS + 1) * 16,), jnp.int32),
                   pltpu.SemaphoreType.DMA],
    compiler_params=pltpu.CompilerParams(needs_layout_passes=False,
                                         use_tc_tiling_on_sc=False),
)
def _bucketize(inter_hbm, srcl_out, dll_out, gath, srcl, dll, cnth, sem):
    _bucket_body(inter_hbm, srcl_out, dll_out, gath, srcl, dll, cnth, sem)


def _segmax_body(b_hbm, srcl_hbm, dll_hbm, s_out, idx, dlb, rowsA, rowsB,
                 rowsC, rowsD, table, semA, semB, semC, semD):
    wid, lo, sz = _wid_lo_sz()
    pltpu.sync_copy(srcl_hbm.at[wid], idx)
    pltpu.sync_copy(dll_hbm.at[wid], dlb)

    neg16 = jnp.full((16,), -jnp.inf, jnp.float32)

    def initb(i, _):
        table[pl.ds(i * 16, 16)] = neg16
        return 0
    lax.fori_loop(0, TABLE_ROWS * 4, initb, 0)

    lane = lax.iota(jnp.int32, 16)
    shift_idx = jnp.maximum(lane - 1, 0)

    bufs = (rowsA, rowsB, rowsC, rowsD)
    sems = (semA, semB, semC, semD)
    for p in range(3):
        pltpu.async_copy(b_hbm.at[idx.at[p]], bufs[p], sems[p])

    def quad(j4, carry):
        for par in range(4):
            j = j4 * 4 + par
            rows = bufs[par]
            sem = sems[par]
            pltpu.make_async_copy(b_hbm.at[idx.at[j]], rows, sem).wait()

            @plsc.parallel_loop(0, 8, unroll=2, carry=carry)
            def grp(g, carry):
                prev, curb, a0, a1, a2, a3 = carry
                acc = [a0, a1, a2, a3]
                dl16 = dlb[pl.ds(j * 128 + g * 16, 16)]
                shifted = jnp.where(lane == 0, _bcast(prev, 15),
                                    _gathv(dl16, shift_idx))
                bst16 = (dl16 != shifted).astype(jnp.int32)
                base16 = dl16 * 64
                for i in range(16):
                    b = _bcast(bst16, i) != 0
                    nb = _bcast(base16, i) + lane
                    e = g * 16 + i
                    rr = []
                    for f2 in range(2):
                        w32 = plsc.bitcast(rows[e, pl.ds(f2 * 32, 32)],
                                           jnp.int32)
                        rr.append(plsc.bitcast(w32 << 16, jnp.float32))
                        rr.append(plsc.bitcast(w32 & -65536, jnp.float32))
                    for f in range(4):
                        plsc.store_scatter(table, [curb + (f * 16)],
                                           acc[f], mask=b)
                        acc[f] = jnp.where(b, rr[f],
                                           jnp.maximum(acc[f], rr[f]))
                    curb = jnp.where(b, nb, curb)
                return (_bcast(dl16, 15), curb, acc[0], acc[1], acc[2], acc[3])

            carry = grp

            @pl.when(j < CHUNKS - 3)
            def _():
                pltpu.async_copy(b_hbm.at[idx.at[j + 3]],
                                 bufs[(par + 3) % 4], sems[(par + 3) % 4])
        return carry

    init = (jnp.full((16,), -1, jnp.int32),
            jnp.full((16,), (TABLE_ROWS - 1) * 64, jnp.int32) + lane,
            neg16, neg16, neg16, neg16)
    prev, curb, a0, a1, a2, a3 = lax.fori_loop(0, CHUNKS // 4, quad, init)
    for f, af in enumerate((a0, a1, a2, a3)):
        plsc.store_scatter(table, [curb + (f * 16)], af)

    @pl.when(wid < 31)
    def _():
        pltpu.sync_copy(table.at[pl.ds(0, 313 * 64)],
                        s_out.at[pl.ds(lo * 64, 313 * 64)])

    @pl.when(wid == 31)
    def _():
        pltpu.sync_copy(table.at[pl.ds(0, 297 * 64)],
                        s_out.at[pl.ds(lo * 64, 297 * 64)])


@functools.partial(
    pl.kernel,
    out_type=jax.ShapeDtypeStruct((N * 64,), jnp.float32),
    mesh=_mesh,
    scratch_types=[pltpu.VMEM((CHUNKS, 128), jnp.int32),
                   pltpu.VMEM((LIST_LEN,), jnp.int32),
                   pltpu.VMEM((128, 64), jnp.bfloat16),
                   pltpu.VMEM((128, 64), jnp.bfloat16),
                   pltpu.VMEM((128, 64), jnp.bfloat16),
                   pltpu.VMEM((128, 64), jnp.bfloat16),
                   pltpu.VMEM((TABLE_ROWS * 64,), jnp.float32),
                   pltpu.SemaphoreType.DMA,
                   pltpu.SemaphoreType.DMA,
                   pltpu.SemaphoreType.DMA,
                   pltpu.SemaphoreType.DMA],
    compiler_params=pltpu.CompilerParams(needs_layout_passes=False,
                                         use_tc_tiling_on_sc=False),
)
def _segmax(b_hbm, srcl_hbm, dll_hbm, s_out, idx, dlb, rowsA, rowsB, rowsC,
            rowsD, table, semA, semB, semC, semD):
    _segmax_body(b_hbm, srcl_hbm, dll_hbm, s_out, idx, dlb, rowsA, rowsB,
                 rowsC, rowsD, table, semA, semB, semC, semD)


_R = 1000


def _lrelu(v):
    return jnp.where(v >= 0, v, 0.01 * v)


def _perm():
    import numpy as _np
    k = _np.arange(16)
    p01 = _np.ravel(_np.column_stack([k, k + 16]))
    return _np.concatenate([p01, p01 + 32])


def _prep_w(w, d):
    wa = w[:, :d]
    wb = w[:, d:]
    return jnp.concatenate([wa - wb, wb[_perm()]], axis=0)


def _tc1_body(x_ref, w_ref, b_ref, a_ref, bv_ref):
    ab = lax.dot_general(x_ref[...], w_ref[...], (((1,), (1,)), ((), ())),
                         preferred_element_type=jnp.float32)
    a_ref[...] = ab[:, :64] + b_ref[...]
    bv_ref[...] = ab[:, 64:].astype(jnp.bfloat16)


def _tc1(x, w, b):
    return pl.pallas_call(
        _tc1_body,
        grid=(N // _R,),
        in_specs=[pl.BlockSpec((_R, 128), lambda i: (i, 0)),
                  pl.BlockSpec((128, 128), lambda i: (0, 0)),
                  pl.BlockSpec((1, 64), lambda i: (0, 0))],
        out_specs=[pl.BlockSpec((_R, 64), lambda i: (i, 0)),
                   pl.BlockSpec((_R, 64), lambda i: (i, 0))],
        out_shape=[jax.ShapeDtypeStruct((N, 64), jnp.float32),
                   jax.ShapeDtypeStruct((N, 64), jnp.bfloat16)],
    )(x, _prep_w(w, 128), b.reshape(1, 64))


def _make_layer_body(nx, d):
    def body(*refs):
        xrefs = refs[:nx]
        aprev, sprev, w_ref, b_ref = refs[nx:nx + 4]
        xout, aout, bvout = refs[nx + 4:]
        v = aprev[...] + sprev[...]
        xp = _lrelu(jnp.where(jnp.isfinite(v), v, 0.0))
        xout[...] = xp
        X = jnp.concatenate([r[...] for r in xrefs] + [xp], axis=1)
        ab = lax.dot_general(X, w_ref[...], (((1,), (1,)), ((), ())),
                             preferred_element_type=jnp.float32)
        aout[...] = ab[:, :64] + b_ref[...]
        bvout[...] = ab[:, 64:].astype(jnp.bfloat16)
    return body


def _tc_layer(xparts, aprev, sprev, w, b):
    widths = [int(p.shape[1]) for p in xparts]
    d = sum(widths) + 64
    in_specs = ([pl.BlockSpec((_R, wd), lambda i: (i, 0)) for wd in widths]
                + [pl.BlockSpec((_R, 64), lambda i: (i, 0)),
                   pl.BlockSpec((_R, 64), lambda i: (i, 0)),
                   pl.BlockSpec((128, d), lambda i: (0, 0)),
                   pl.BlockSpec((1, 64), lambda i: (0, 0))])
    return pl.pallas_call(
        _make_layer_body(len(xparts), d),
        grid=(N // _R,),
        in_specs=in_specs,
        out_specs=[pl.BlockSpec((_R, 64), lambda i: (i, 0))] * 3,
        out_shape=[jax.ShapeDtypeStruct((N, 64), jnp.float32),
                   jax.ShapeDtypeStruct((N, 64), jnp.float32),
                   jax.ShapeDtypeStruct((N, 64), jnp.bfloat16)],
    )(*xparts, aprev, sprev, _prep_w(w, d), b.reshape(1, 64))


def _tcf_body(x0, x1, x2, x3, a4, s4, wm1, bm1, wm2, bm2, out):
    v = a4[...] + s4[...]
    x4 = _lrelu(jnp.where(jnp.isfinite(v), v, 0.0))
    F = jnp.concatenate([x0[...], x1[...], x2[...], x3[...], x4], axis=1)
    h = lax.dot_general(F, wm1[...], (((1,), (1,)), ((), ())),
                        preferred_element_type=jnp.float32) + bm1[...]
    out[...] = lax.dot_general(h, wm2[...], (((1,), (1,)), ((), ())),
                               preferred_element_type=jnp.float32) + bm2[...]


def _tc_final(x0, x1, x2, x3, a4, s4, wm1, bm1, wm2, bm2):
    return pl.pallas_call(
        _tcf_body,
        grid=(N // _R,),
        in_specs=[pl.BlockSpec((_R, 128), lambda i: (i, 0)),
                  pl.BlockSpec((_R, 64), lambda i: (i, 0)),
                  pl.BlockSpec((_R, 64), lambda i: (i, 0)),
                  pl.BlockSpec((_R, 64), lambda i: (i, 0)),
                  pl.BlockSpec((_R, 64), lambda i: (i, 0)),
                  pl.BlockSpec((_R, 64), lambda i: (i, 0)),
                  pl.BlockSpec((64, 384), lambda i: (0, 0)),
                  pl.BlockSpec((1, 64), lambda i: (0, 0)),
                  pl.BlockSpec((10, 64), lambda i: (0, 0)),
                  pl.BlockSpec((1, 10), lambda i: (0, 0))],
        out_specs=pl.BlockSpec((_R, 10), lambda i: (i, 0)),
        out_shape=jax.ShapeDtypeStruct((N, 10), jnp.float32),
    )(x0, x1, x2, x3, a4, s4, wm1, bm1.reshape(1, 64), wm2,
      bm2.reshape(1, 10))


def kernel(x, edge_index, W1, b1, W2, b2, W3, b3, W4, b4, Wm1, bm1, Wm2, bm2):
    src = edge_index[0]
    dst = edge_index[1]
    inter = _phase_a(src, dst)
    srcl, dll = _bucketize(inter)
    srcl3 = srcl.reshape(NW, CHUNKS, 128)

    a1, b1v = _tc1(x, W1, b1)
    s1 = _segmax(b1v, srcl3, dll).reshape(N, 64)
    x1, a2, b2v = _tc_layer([x], a1, s1, W2, b2)
    s2 = _segmax(b2v, srcl3, dll).reshape(N, 64)
    x2, a3, b3v = _tc_layer([x, x1], a2, s2, W3, b3)
    s3 = _segmax(b3v, srcl3, dll).reshape(N, 64)
    x3, a4, b4v = _tc_layer([x, x1, x2], a3, s3, W4, b4)
    s4 = _segmax(b4v, srcl3, dll).reshape(N, 64)
    return _tc_final(x, x1, x2, x3, a4, s4, Wm1, bm1, Wm2, bm2)
